# baseline (speedup 1.0000x reference)
import functools

import jax
import jax.numpy as jnp
from jax import lax
from jax.experimental import pallas as pl
from jax.experimental.pallas import tpu as pltpu
from jax.experimental.pallas import tpu_sc as plsc

N = 10000
E = 320000
D = 128
H = 128
G = 16
EPS = 1e-5

NPAD = 10240
ACC = 2 * NPAD
ROWS = 2816
EA = ROWS * 128
PADCNT = EA - E - 2 * N
NW = 32
RPW = ROWS // NW
RPS = ROWS // 16
CR = 8
STRIPE = ACC // 16

_MESH = plsc.VectorSubcoreMesh(
    core_axis_name="c", subcore_axis_name="s", num_cores=2, num_subcores=16)


def _deg_body(dst_hbm, ew_hbm, z_hbm, out_hbm, acc, dst_v, ew_v, idx_v, w_v):
    c = lax.axis_index("c")
    s = lax.axis_index("s")
    w = s * 2 + c
    pltpu.sync_copy(z_hbm, acc.at[pl.ds(s * STRIPE, STRIPE)])
    plsc.subcore_barrier()

    def chunk(ch, _):
        base = w * RPW + ch * CR
        pltpu.sync_copy(dst_hbm.at[pl.ds(base, CR)], dst_v)
        pltpu.sync_copy(ew_hbm.at[pl.ds(base, CR)], ew_v)

        def grp(t, _):
            j = t // 8
            k = (t % 8) * 16
            ew = ew_v[j, pl.ds(k, 16)]
            dv = dst_v[j, pl.ds(k, 16)]
            off = jnp.where(ew < 0.0, jnp.int32(NPAD), jnp.int32(0))
            idx_v[j, pl.ds(k, 16)] = dv + off
            w_v[j, pl.ds(k, 16)] = jnp.abs(ew)
            return 0

        lax.fori_loop(0, CR * 8, grp, 0)

        def scat(j, _):
            pltpu.sync_copy(w_v.at[j], acc.at[idx_v.at[j]], add=True)
            return 0

        lax.fori_loop(0, CR, scat, 0)
        return 0

    lax.fori_loop(0, RPW // CR, chunk, 0)
    plsc.subcore_barrier()
    pltpu.sync_copy(acc.at[pl.ds(s * STRIPE, STRIPE)],
                    out_hbm.at[pl.ds(c * ACC + s * STRIPE, STRIPE)])


_deg_call = functools.partial(
    pl.kernel,
    _deg_body,
    out_type=jax.ShapeDtypeStruct((2 * ACC,), jnp.float32),
    mesh=_MESH,
    scratch_types=[
        pltpu.VMEM_SHARED((ACC,), jnp.float32),
        pltpu.VMEM((CR, 128), jnp.int32),
        pltpu.VMEM((CR, 128), jnp.float32),
        pltpu.VMEM((CR, 128), jnp.int32),
        pltpu.VMEM((CR, 128), jnp.float32),
    ],
    compiler_params=pltpu.CompilerParams(needs_layout_passes=False, use_tc_tiling_on_sc=False),
)()


def _norm_body(src_hbm, dst_hbm, ew_hbm, cdis_hbm, cn_hbm, ci_hbm,
               cdis_v, src_v, dst_v, ew_v, cn_v, ci_v):
    c = lax.axis_index("c")
    s = lax.axis_index("s")
    w = s * 2 + c
    pltpu.sync_copy(cdis_hbm, cdis_v)

    def chunk(ch, _):
        base = w * RPW + ch * CR
        pltpu.sync_copy(src_hbm.at[pl.ds(base, CR)], src_v)
        pltpu.sync_copy(dst_hbm.at[pl.ds(base, CR)], dst_v)
        pltpu.sync_copy(ew_hbm.at[pl.ds(base, CR)], ew_v)

        def grp(t, _):
            j = t // 8
            k = (t % 8) * 16
            ew = ew_v[j, pl.ds(k, 16)]
            sv = src_v[j, pl.ds(k, 16)]
            dv = dst_v[j, pl.ds(k, 16)]
            off = jnp.where(ew < 0.0, jnp.int32(NPAD), jnp.int32(0))
            gs = plsc.load_gather(cdis_v, [sv + off])
            gd = plsc.load_gather(cdis_v, [dv + off])
            cn_v[j, pl.ds(k, 16)] = gs * gd * jnp.abs(ew)
            ci_v[j, pl.ds(k, 16)] = dv + off
            return 0

        lax.fori_loop(0, CR * 8, grp, 0)
        pltpu.sync_copy(cn_v, cn_hbm.at[pl.ds(base, CR)])
        pltpu.sync_copy(ci_v, ci_hbm.at[pl.ds(base, CR)])
        return 0

    lax.fori_loop(0, RPW // CR, chunk, 0)


_norm_call = functools.partial(
    pl.kernel,
    _norm_body,
    out_type=(jax.ShapeDtypeStruct((ROWS, 128), jnp.float32),
              jax.ShapeDtypeStruct((ROWS, 128), jnp.int32)),
    mesh=_MESH,
    scratch_types=[
        pltpu.VMEM((ACC,), jnp.float32),
        pltpu.VMEM((CR, 128), jnp.int32),
        pltpu.VMEM((CR, 128), jnp.int32),
        pltpu.VMEM((CR, 128), jnp.float32),
        pltpu.VMEM((CR, 128), jnp.float32),
        pltpu.VMEM((CR, 128), jnp.int32),
    ],
    compiler_params=pltpu.CompilerParams(needs_layout_passes=False, use_tc_tiling_on_sc=False),
)()


CRA = 16


def _agg_body(src_hbm, ci_hbm, cn_hbm, tab_hbm, z_hbm, out_hbm,
              acc, src_v, gi_v, ci_v, cn_v, buf0, buf1, sem0, sem1):
    c = lax.axis_index("c")
    s = lax.axis_index("s")
    pltpu.sync_copy(z_hbm, acc.at[pl.ds(s * STRIPE, STRIPE)])
    plsc.subcore_barrier()
    toff = c * jnp.int32(N)
    bufs = (buf0, buf1)
    sems = (sem0, sem1)

    def chunk(ch, _):
        base = s * RPS + ch * CRA
        pltpu.sync_copy(src_hbm.at[pl.ds(base, CRA)], src_v)
        pltpu.sync_copy(ci_hbm.at[pl.ds(base, CRA)], ci_v)
        pltpu.sync_copy(cn_hbm.at[pl.ds(base, CRA)], cn_v)

        def grp(t, _):
            j = t // 8
            k = (t % 8) * 16
            gi_v[j, pl.ds(k, 16)] = src_v[j, pl.ds(k, 16)] + toff
            return 0

        lax.fori_loop(0, CRA * 8, grp, 0)

        for j in range(CRA):
            b = j % 2
            buf = bufs[b]
            pltpu.sync_copy(tab_hbm.at[gi_v.at[j]], buf)

            def egrp(g, _, j=j, buf=buf):
                cnvec = cn_v[j, pl.ds(g * 16, 16)]
                for l in range(16):
                    e = g * 16 + l
                    cn = cnvec[l]
                    for k in range(4):
                        buf[e, pl.ds(k * 16, 16)] = (
                            buf[e, pl.ds(k * 16, 16)] * cn)
                return 0

            lax.fori_loop(0, 8, egrp, 0)
            pltpu.sync_copy(buf, acc.at[ci_v.at[j]], add=True)
        return 0

    lax.fori_loop(0, RPS // CRA, chunk, 0)
    plsc.subcore_barrier()
    pltpu.sync_copy(acc.at[pl.ds(s * STRIPE, STRIPE)],
                    out_hbm.at[pl.ds(c * ACC + s * STRIPE, STRIPE)])


_agg_call = functools.partial(
    pl.kernel,
    _agg_body,
    out_type=jax.ShapeDtypeStruct((2 * ACC, 64), jnp.float32),
    mesh=_MESH,
    scratch_types=[
        pltpu.VMEM_SHARED((ACC, 64), jnp.float32),
        pltpu.VMEM((CRA, 128), jnp.int32),
        pltpu.VMEM((CRA, 128), jnp.int32),
        pltpu.VMEM((CRA, 128), jnp.int32),
        pltpu.VMEM((CRA, 128), jnp.float32),
        pltpu.VMEM((128, 64), jnp.float32),
        pltpu.VMEM((128, 64), jnp.float32),
        pltpu.SemaphoreType.DMA,
        pltpu.SemaphoreType.DMA,
    ],
    compiler_params=pltpu.CompilerParams(needs_layout_passes=False, use_tc_tiling_on_sc=False),
)()


def _prep_kernel(deg_ref, o_ref):
    d = deg_ref[0] + deg_ref[1]
    o_ref[...] = jnp.where(d > 0.0, lax.rsqrt(jnp.maximum(d, 1e-30)), 0.0)


def _transform_kernel(pa, pb, na, nb, wpa, wpb, wna, wnb, bp, bn, oa, ob):
    dn = (((1,), (0,)), ((), ()))
    px = (lax.dot_general(pa[...], wpa[...], dn, precision=lax.Precision.HIGHEST)
          + lax.dot_general(pb[...], wpb[...], dn, precision=lax.Precision.HIGHEST)
          + bp[...])
    nx = (lax.dot_general(na[...], wna[...], dn, precision=lax.Precision.HIGHEST)
          + lax.dot_general(nb[...], wnb[...], dn, precision=lax.Precision.HIGHEST)
          + bn[...])
    h = jnp.maximum(px, 0.0) - jnp.maximum(nx, 0.0)
    oa[...] = h[:, :64]
    ob[...] = h[:, 64:]


def _final_kernel(pa, pb, na, nb, wpa, wpb, wna, wnb, bp, bn, bat, gam, bet,
                  o_ref, acc, cnt):
    i = pl.program_id(0)

    @pl.when(i == 0)
    def _():
        acc[...] = jnp.zeros_like(acc)
        cnt[...] = jnp.zeros_like(cnt)

    dn = (((1,), (0,)), ((), ()))
    dnp = (((0,), (0,)), ((), ()))
    px = (lax.dot_general(pa[...], wpa[...], dn, precision=lax.Precision.HIGHEST)
          + lax.dot_general(pb[...], wpb[...], dn, precision=lax.Precision.HIGHEST)
          + bp[...])
    nx = (lax.dot_general(na[...], wna[...], dn, precision=lax.Precision.HIGHEST)
          + lax.dot_general(nb[...], wnb[...], dn, precision=lax.Precision.HIGHEST)
          + bn[...])
    h = jnp.maximum(px, 0.0) - jnp.maximum(nx, 0.0)
    oh = (bat[...] == lax.broadcasted_iota(jnp.int32, (400, G), 1
                                           ).astype(jnp.float32)
          ).astype(jnp.float32)
    acc[...] += lax.dot_general(oh, h, dnp, precision=lax.Precision.HIGHEST)
    cnt[...] += lax.dot_general(oh, jnp.ones_like(h), dnp,
                                precision=lax.Precision.HIGHEST)

    @pl.when(i == pl.num_programs(0) - 1)
    def _():
        pooled = acc[...] / jnp.maximum(cnt[...], 1.0)
        mu = jnp.mean(pooled, axis=-1, keepdims=True)
        dev = pooled - mu
        var = jnp.mean(dev * dev, axis=-1, keepdims=True)
        o_ref[...] = dev * lax.rsqrt(var + EPS) * gam[...] + bet[...]


def _transform(aggf, Wp, bp, Wn, bn):
    pa = aggf[0:N]
    na = aggf[NPAD:NPAD + N]
    pb = aggf[ACC:ACC + N]
    nb = aggf[ACC + NPAD:ACC + NPAD + N]
    bn_ = 400
    grid = N // bn_
    rspec = pl.BlockSpec((bn_, 64), lambda i: (i, 0))
    wspec = pl.BlockSpec((64, 128), lambda i: (0, 0))
    bspec = pl.BlockSpec((1, 128), lambda i: (0, 0))
    return pl.pallas_call(
        _transform_kernel,
        grid=(grid,),
        in_specs=[rspec, rspec, rspec, rspec,
                  wspec, wspec, wspec, wspec, bspec, bspec],
        out_specs=[pl.BlockSpec((bn_, 64), lambda i: (i, 0))] * 2,
        out_shape=[jax.ShapeDtypeStruct((N, 64), jnp.float32)] * 2,
        compiler_params=pltpu.CompilerParams(
            dimension_semantics=("arbitrary",)),
    )(pa, pb, na, nb, Wp[:64], Wp[64:], Wn[:64], Wn[64:],
      bp.reshape(1, 128), bn.reshape(1, 128))


def _final(aggf, Wp, bp, Wn, bn, batf, gamma, beta):
    pa = aggf[0:N]
    na = aggf[NPAD:NPAD + N]
    pb = aggf[ACC:ACC + N]
    nb = aggf[ACC + NPAD:ACC + NPAD + N]
    bn_ = 400
    grid = N // bn_
    rspec = pl.BlockSpec((bn_, 64), lambda i: (i, 0))
    wspec = pl.BlockSpec((64, 128), lambda i: (0, 0))
    bspec = pl.BlockSpec((1, 128), lambda i: (0, 0))
    return pl.pallas_call(
        _final_kernel,
        grid=(grid,),
        in_specs=[rspec, rspec, rspec, rspec,
                  wspec, wspec, wspec, wspec, bspec, bspec,
                  pl.BlockSpec((bn_, 1), lambda i: (i, 0)),
                  bspec, bspec],
        out_specs=pl.BlockSpec((G, 128), lambda i: (0, 0)),
        out_shape=jax.ShapeDtypeStruct((G, 128), jnp.float32),
        scratch_shapes=[pltpu.VMEM((G, 128), jnp.float32),
                        pltpu.VMEM((G, 128), jnp.float32)],
        compiler_params=pltpu.CompilerParams(
            dimension_semantics=("arbitrary",)),
    )(pa, pb, na, nb, Wp[:64], Wp[64:], Wn[:64], Wn[64:],
      bp.reshape(1, 128), bn.reshape(1, 128), batf,
      gamma.reshape(1, 128), beta.reshape(1, 128))


def kernel(x, edge_index, edge_weight, batch,
           Wp0, bp0, Wp1, bp1, Wn0, bn0, Wn1, bn1, gamma, beta):
    ar = jnp.arange(N, dtype=jnp.int32)
    padi = jnp.arange(PADCNT, dtype=jnp.int32) % N
    src = jnp.concatenate([edge_index[0], ar, ar, padi]).reshape(ROWS, 128)
    dst = jnp.concatenate([edge_index[1], ar, ar, padi]).reshape(ROWS, 128)
    ew = jnp.concatenate([
        edge_weight, jnp.ones((N,), jnp.float32),
        -jnp.ones((N,), jnp.float32), jnp.zeros((PADCNT,), jnp.float32),
    ]).reshape(ROWS, 128)
    zdeg = jnp.zeros((STRIPE,), jnp.float32)
    zagg = jnp.zeros((STRIPE, 64), jnp.float32)

    degf = _deg_call(dst, ew, zdeg)
    cdis = pl.pallas_call(
        _prep_kernel,
        out_shape=jax.ShapeDtypeStruct((ACC // 128, 128), jnp.float32),
    )(degf.reshape(2, ACC // 128, 128)).reshape(ACC)
    cn2d, ci2d = _norm_call(src, dst, ew, cdis)

    tab1 = jnp.concatenate([x[:, :64], x[:, 64:]], axis=0)
    agg1 = _agg_call(src, ci2d, cn2d, tab1, zagg)
    h1a, h1b = _transform(agg1, Wp0, bp0, Wn0, bn0)

    tab2 = jnp.concatenate([h1a, h1b], axis=0)
    agg2 = _agg_call(src, ci2d, cn2d, tab2, zagg)

    batf = batch.astype(jnp.float32).reshape(N, 1)
    return _final(agg2, Wp1, bp1, Wn1, bn1, batf, gamma, beta)


# speedup vs baseline: 2.1315x; 2.1315x over previous
import functools

import jax
import jax.numpy as jnp
from jax import lax
from jax.experimental import pallas as pl
from jax.experimental.pallas import tpu as pltpu
from jax.experimental.pallas import tpu_sc as plsc

N = 10000
E = 320000
D = 128
H = 128
G = 16
EPS = 1e-5

NPAD = 10240
ACC = 2 * NPAD
ROWS = 2816
EA = ROWS * 128
PADCNT = EA - E - 2 * N
NW = 32
RPW = ROWS // NW
RPS = ROWS // 16
CR = 8
STRIPE = ACC // 16

_MESH = plsc.VectorSubcoreMesh(
    core_axis_name="c", subcore_axis_name="s", num_cores=2, num_subcores=16)


def _deg_body(dst_hbm, ew_hbm, z_hbm, out_hbm, acc, dst_v, ew_v, idx_v, w_v):
    c = lax.axis_index("c")
    s = lax.axis_index("s")
    w = s * 2 + c
    pltpu.sync_copy(z_hbm, acc.at[pl.ds(s * STRIPE, STRIPE)])
    plsc.subcore_barrier()

    def chunk(ch, _):
        base = w * RPW + ch * CR
        pltpu.sync_copy(dst_hbm.at[pl.ds(base, CR)], dst_v)
        pltpu.sync_copy(ew_hbm.at[pl.ds(base, CR)], ew_v)

        def grp(t, _):
            j = t // 8
            k = (t % 8) * 16
            ew = ew_v[j, pl.ds(k, 16)]
            dv = dst_v[j, pl.ds(k, 16)]
            off = jnp.where(ew < 0.0, jnp.int32(NPAD), jnp.int32(0))
            idx_v[j, pl.ds(k, 16)] = dv + off
            w_v[j, pl.ds(k, 16)] = jnp.abs(ew)
            return 0

        lax.fori_loop(0, CR * 8, grp, 0)

        def scat(j, _):
            pltpu.sync_copy(w_v.at[j], acc.at[idx_v.at[j]], add=True)
            return 0

        lax.fori_loop(0, CR, scat, 0)
        return 0

    lax.fori_loop(0, RPW // CR, chunk, 0)
    plsc.subcore_barrier()
    pltpu.sync_copy(acc.at[pl.ds(s * STRIPE, STRIPE)],
                    out_hbm.at[pl.ds(c * ACC + s * STRIPE, STRIPE)])


_deg_call = functools.partial(
    pl.kernel,
    _deg_body,
    out_type=jax.ShapeDtypeStruct((2 * ACC,), jnp.float32),
    mesh=_MESH,
    scratch_types=[
        pltpu.VMEM_SHARED((ACC,), jnp.float32),
        pltpu.VMEM((CR, 128), jnp.int32),
        pltpu.VMEM((CR, 128), jnp.float32),
        pltpu.VMEM((CR, 128), jnp.int32),
        pltpu.VMEM((CR, 128), jnp.float32),
    ],
    compiler_params=pltpu.CompilerParams(needs_layout_passes=False, use_tc_tiling_on_sc=False),
)()


def _norm_body(src_hbm, dst_hbm, ew_hbm, cdis_hbm, cn_hbm, ci_hbm,
               cdis_v, src_v, dst_v, ew_v, cn_v, ci_v):
    c = lax.axis_index("c")
    s = lax.axis_index("s")
    w = s * 2 + c
    pltpu.sync_copy(cdis_hbm, cdis_v)

    def chunk(ch, _):
        base = w * RPW + ch * CR
        pltpu.sync_copy(src_hbm.at[pl.ds(base, CR)], src_v)
        pltpu.sync_copy(dst_hbm.at[pl.ds(base, CR)], dst_v)
        pltpu.sync_copy(ew_hbm.at[pl.ds(base, CR)], ew_v)

        def grp(t, _):
            j = t // 8
            k = (t % 8) * 16
            ew = ew_v[j, pl.ds(k, 16)]
            sv = src_v[j, pl.ds(k, 16)]
            dv = dst_v[j, pl.ds(k, 16)]
            off = jnp.where(ew < 0.0, jnp.int32(NPAD), jnp.int32(0))
            gs = plsc.load_gather(cdis_v, [sv + off])
            gd = plsc.load_gather(cdis_v, [dv + off])
            cn_v[j, pl.ds(k, 16)] = gs * gd * jnp.abs(ew)
            ci_v[j, pl.ds(k, 16)] = dv + off
            return 0

        lax.fori_loop(0, CR * 8, grp, 0)
        pltpu.sync_copy(cn_v, cn_hbm.at[pl.ds(base, CR)])
        pltpu.sync_copy(ci_v, ci_hbm.at[pl.ds(base, CR)])
        return 0

    lax.fori_loop(0, RPW // CR, chunk, 0)


_norm_call = functools.partial(
    pl.kernel,
    _norm_body,
    out_type=(jax.ShapeDtypeStruct((ROWS, 128), jnp.float32),
              jax.ShapeDtypeStruct((ROWS, 128), jnp.int32)),
    mesh=_MESH,
    scratch_types=[
        pltpu.VMEM((ACC,), jnp.float32),
        pltpu.VMEM((CR, 128), jnp.int32),
        pltpu.VMEM((CR, 128), jnp.int32),
        pltpu.VMEM((CR, 128), jnp.float32),
        pltpu.VMEM((CR, 128), jnp.float32),
        pltpu.VMEM((CR, 128), jnp.int32),
    ],
    compiler_params=pltpu.CompilerParams(needs_layout_passes=False, use_tc_tiling_on_sc=False),
)()


def _agg_body(src_hbm, ci_hbm, cn_hbm, tab_hbm, z_hbm, out_hbm,
              acc, src_v, gi_v, ci_v, cn_v, buf0, buf1, msg_v, sem0, sem1):
    c = lax.axis_index("c")
    s = lax.axis_index("s")
    pltpu.sync_copy(z_hbm, acc.at[pl.ds(s * STRIPE, STRIPE)])
    plsc.subcore_barrier()
    toff = c * jnp.int32(N)

    def scale_scatter(buf, j):
        def egrp(g, _):
            cnvec = cn_v[j, pl.ds(g * 16, 16)]
            for l in range(16):
                e = g * 16 + l
                cn = cnvec[l]
                for k in range(4):
                    msg_v[e, pl.ds(k * 16, 16)] = (
                        buf[e, pl.ds(k * 16, 16)] * cn)
            return 0

        lax.fori_loop(0, 8, egrp, 0)
        pltpu.sync_copy(msg_v, acc.at[ci_v.at[j]], add=True)

    def chunk(ch, _):
        base = s * RPS + ch * CR
        pltpu.sync_copy(src_hbm.at[pl.ds(base, CR)], src_v)
        pltpu.sync_copy(ci_hbm.at[pl.ds(base, CR)], ci_v)
        pltpu.sync_copy(cn_hbm.at[pl.ds(base, CR)], cn_v)

        def grp(t, _):
            j = t // 8
            k = (t % 8) * 16
            gi_v[j, pl.ds(k, 16)] = src_v[j, pl.ds(k, 16)] + toff
            return 0

        lax.fori_loop(0, CR * 8, grp, 0)

        pltpu.async_copy(tab_hbm.at[gi_v.at[0]], buf0, sem0)

        def pair(it, _):
            jj = 2 * it
            pltpu.async_copy(tab_hbm.at[gi_v.at[jj + 1]], buf1, sem1)
            pltpu.make_async_copy(tab_hbm.at[gi_v.at[0]], buf0, sem0).wait()
            scale_scatter(buf0, jj)

            @pl.when(it + 1 < CR // 2)
            def _():
                pltpu.async_copy(tab_hbm.at[gi_v.at[jj + 2]], buf0, sem0)

            pltpu.make_async_copy(tab_hbm.at[gi_v.at[0]], buf1, sem1).wait()
            scale_scatter(buf1, jj + 1)
            return 0

        lax.fori_loop(0, CR // 2, pair, 0)
        return 0

    lax.fori_loop(0, RPS // CR, chunk, 0)
    plsc.subcore_barrier()
    pltpu.sync_copy(acc.at[pl.ds(s * STRIPE, STRIPE)],
                    out_hbm.at[pl.ds(c * ACC + s * STRIPE, STRIPE)])


_agg_call = functools.partial(
    pl.kernel,
    _agg_body,
    out_type=jax.ShapeDtypeStruct((2 * ACC, 64), jnp.float32),
    mesh=_MESH,
    scratch_types=[
        pltpu.VMEM_SHARED((ACC, 64), jnp.float32),
        pltpu.VMEM((CR, 128), jnp.int32),
        pltpu.VMEM((CR, 128), jnp.int32),
        pltpu.VMEM((CR, 128), jnp.int32),
        pltpu.VMEM((CR, 128), jnp.float32),
        pltpu.VMEM((128, 64), jnp.float32),
        pltpu.VMEM((128, 64), jnp.float32),
        pltpu.VMEM((128, 64), jnp.float32),
        pltpu.SemaphoreType.DMA,
        pltpu.SemaphoreType.DMA,
    ],
    compiler_params=pltpu.CompilerParams(needs_layout_passes=False, use_tc_tiling_on_sc=False),
)()


def _prep_kernel(deg_ref, o_ref):
    d = deg_ref[0] + deg_ref[1]
    o_ref[...] = jnp.where(d > 0.0, lax.rsqrt(jnp.maximum(d, 1e-30)), 0.0)


def _transform_kernel(pa, pb, na, nb, wpa, wpb, wna, wnb, bp, bn, oa, ob):
    dn = (((1,), (0,)), ((), ()))
    px = (lax.dot_general(pa[...], wpa[...], dn, precision=lax.Precision.HIGHEST)
          + lax.dot_general(pb[...], wpb[...], dn, precision=lax.Precision.HIGHEST)
          + bp[...])
    nx = (lax.dot_general(na[...], wna[...], dn, precision=lax.Precision.HIGHEST)
          + lax.dot_general(nb[...], wnb[...], dn, precision=lax.Precision.HIGHEST)
          + bn[...])
    h = jnp.maximum(px, 0.0) - jnp.maximum(nx, 0.0)
    oa[...] = h[:, :64]
    ob[...] = h[:, 64:]


def _final_kernel(pa, pb, na, nb, wpa, wpb, wna, wnb, bp, bn, bat, gam, bet,
                  o_ref, acc, cnt):
    i = pl.program_id(0)

    @pl.when(i == 0)
    def _():
        acc[...] = jnp.zeros_like(acc)
        cnt[...] = jnp.zeros_like(cnt)

    dn = (((1,), (0,)), ((), ()))
    dnp = (((0,), (0,)), ((), ()))
    px = (lax.dot_general(pa[...], wpa[...], dn, precision=lax.Precision.HIGHEST)
          + lax.dot_general(pb[...], wpb[...], dn, precision=lax.Precision.HIGHEST)
          + bp[...])
    nx = (lax.dot_general(na[...], wna[...], dn, precision=lax.Precision.HIGHEST)
          + lax.dot_general(nb[...], wnb[...], dn, precision=lax.Precision.HIGHEST)
          + bn[...])
    h = jnp.maximum(px, 0.0) - jnp.maximum(nx, 0.0)
    oh = (bat[...] == lax.broadcasted_iota(jnp.int32, (400, G), 1
                                           ).astype(jnp.float32)
          ).astype(jnp.float32)
    acc[...] += lax.dot_general(oh, h, dnp, precision=lax.Precision.HIGHEST)
    cnt[...] += lax.dot_general(oh, jnp.ones_like(h), dnp,
                                precision=lax.Precision.HIGHEST)

    @pl.when(i == pl.num_programs(0) - 1)
    def _():
        pooled = acc[...] / jnp.maximum(cnt[...], 1.0)
        mu = jnp.mean(pooled, axis=-1, keepdims=True)
        dev = pooled - mu
        var = jnp.mean(dev * dev, axis=-1, keepdims=True)
        o_ref[...] = dev * lax.rsqrt(var + EPS) * gam[...] + bet[...]


def _transform(aggf, Wp, bp, Wn, bn):
    pa = aggf[0:N]
    na = aggf[NPAD:NPAD + N]
    pb = aggf[ACC:ACC + N]
    nb = aggf[ACC + NPAD:ACC + NPAD + N]
    bn_ = 400
    grid = N // bn_
    rspec = pl.BlockSpec((bn_, 64), lambda i: (i, 0))
    wspec = pl.BlockSpec((64, 128), lambda i: (0, 0))
    bspec = pl.BlockSpec((1, 128), lambda i: (0, 0))
    return pl.pallas_call(
        _transform_kernel,
        grid=(grid,),
        in_specs=[rspec, rspec, rspec, rspec,
                  wspec, wspec, wspec, wspec, bspec, bspec],
        out_specs=[pl.BlockSpec((bn_, 64), lambda i: (i, 0))] * 2,
        out_shape=[jax.ShapeDtypeStruct((N, 64), jnp.float32)] * 2,
        compiler_params=pltpu.CompilerParams(
            dimension_semantics=("arbitrary",)),
    )(pa, pb, na, nb, Wp[:64], Wp[64:], Wn[:64], Wn[64:],
      bp.reshape(1, 128), bn.reshape(1, 128))


def _final(aggf, Wp, bp, Wn, bn, batf, gamma, beta):
    pa = aggf[0:N]
    na = aggf[NPAD:NPAD + N]
    pb = aggf[ACC:ACC + N]
    nb = aggf[ACC + NPAD:ACC + NPAD + N]
    bn_ = 400
    grid = N // bn_
    rspec = pl.BlockSpec((bn_, 64), lambda i: (i, 0))
    wspec = pl.BlockSpec((64, 128), lambda i: (0, 0))
    bspec = pl.BlockSpec((1, 128), lambda i: (0, 0))
    return pl.pallas_call(
        _final_kernel,
        grid=(grid,),
        in_specs=[rspec, rspec, rspec, rspec,
                  wspec, wspec, wspec, wspec, bspec, bspec,
                  pl.BlockSpec((bn_, 1), lambda i: (i, 0)),
                  bspec, bspec],
        out_specs=pl.BlockSpec((G, 128), lambda i: (0, 0)),
        out_shape=jax.ShapeDtypeStruct((G, 128), jnp.float32),
        scratch_shapes=[pltpu.VMEM((G, 128), jnp.float32),
                        pltpu.VMEM((G, 128), jnp.float32)],
        compiler_params=pltpu.CompilerParams(
            dimension_semantics=("arbitrary",)),
    )(pa, pb, na, nb, Wp[:64], Wp[64:], Wn[:64], Wn[64:],
      bp.reshape(1, 128), bn.reshape(1, 128), batf,
      gamma.reshape(1, 128), beta.reshape(1, 128))


def kernel(x, edge_index, edge_weight, batch,
           Wp0, bp0, Wp1, bp1, Wn0, bn0, Wn1, bn1, gamma, beta):
    ar = jnp.arange(N, dtype=jnp.int32)
    padi = jnp.arange(PADCNT, dtype=jnp.int32) % N
    src = jnp.concatenate([edge_index[0], ar, ar, padi]).reshape(ROWS, 128)
    dst = jnp.concatenate([edge_index[1], ar, ar, padi]).reshape(ROWS, 128)
    ew = jnp.concatenate([
        edge_weight, jnp.ones((N,), jnp.float32),
        -jnp.ones((N,), jnp.float32), jnp.zeros((PADCNT,), jnp.float32),
    ]).reshape(ROWS, 128)
    zdeg = jnp.zeros((STRIPE,), jnp.float32)
    zagg = jnp.zeros((STRIPE, 64), jnp.float32)

    degf = _deg_call(dst, ew, zdeg)
    cdis = pl.pallas_call(
        _prep_kernel,
        out_shape=jax.ShapeDtypeStruct((ACC // 128, 128), jnp.float32),
    )(degf.reshape(2, ACC // 128, 128)).reshape(ACC)
    cn2d, ci2d = _norm_call(src, dst, ew, cdis)

    tab1 = jnp.concatenate([x[:, :64], x[:, 64:]], axis=0)
    agg1 = _agg_call(src, ci2d, cn2d, tab1, zagg)
    h1a, h1b = _transform(agg1, Wp0, bp0, Wn0, bn0)

    tab2 = jnp.concatenate([h1a, h1b], axis=0)
    agg2 = _agg_call(src, ci2d, cn2d, tab2, zagg)

    batf = batch.astype(jnp.float32).reshape(N, 1)
    return _final(agg2, Wp1, bp1, Wn1, bn1, batf, gamma, beta)


# speedup vs baseline: 2.3003x; 1.0792x over previous
import functools

import jax
import jax.numpy as jnp
from jax import lax
from jax.experimental import pallas as pl
from jax.experimental.pallas import tpu as pltpu
from jax.experimental.pallas import tpu_sc as plsc

N = 10000
E = 320000
D = 128
H = 128
G = 16
EPS = 1e-5

NPAD = 10240
ACC = 2 * NPAD
ROWS = 2816
EA = ROWS * 128
PADCNT = EA - E - 2 * N
NW = 32
RPW = ROWS // NW
RPS = ROWS // 16
CR = 8
STRIPE = ACC // 16

_MESH = plsc.VectorSubcoreMesh(
    core_axis_name="c", subcore_axis_name="s", num_cores=2, num_subcores=16)


def _deg_body(dst_hbm, ew_hbm, z_hbm, out_hbm, acc, dst_v, ew_v, idx_v, w_v):
    c = lax.axis_index("c")
    s = lax.axis_index("s")
    w = s * 2 + c
    pltpu.sync_copy(z_hbm, acc.at[pl.ds(s * STRIPE, STRIPE)])
    plsc.subcore_barrier()

    def chunk(ch, _):
        base = w * RPW + ch * CR
        pltpu.sync_copy(dst_hbm.at[pl.ds(base, CR)], dst_v)
        pltpu.sync_copy(ew_hbm.at[pl.ds(base, CR)], ew_v)

        def grp(t, _):
            j = t // 8
            k = (t % 8) * 16
            ew = ew_v[j, pl.ds(k, 16)]
            dv = dst_v[j, pl.ds(k, 16)]
            off = jnp.where(ew < 0.0, jnp.int32(NPAD), jnp.int32(0))
            idx_v[j, pl.ds(k, 16)] = dv + off
            w_v[j, pl.ds(k, 16)] = jnp.abs(ew)
            return 0

        lax.fori_loop(0, CR * 8, grp, 0)

        def scat(j, _):
            pltpu.sync_copy(w_v.at[j], acc.at[idx_v.at[j]], add=True)
            return 0

        lax.fori_loop(0, CR, scat, 0)
        return 0

    lax.fori_loop(0, RPW // CR, chunk, 0)
    plsc.subcore_barrier()
    pltpu.sync_copy(acc.at[pl.ds(s * STRIPE, STRIPE)],
                    out_hbm.at[pl.ds(c * ACC + s * STRIPE, STRIPE)])


_deg_call = functools.partial(
    pl.kernel,
    _deg_body,
    out_type=jax.ShapeDtypeStruct((2 * ACC,), jnp.float32),
    mesh=_MESH,
    scratch_types=[
        pltpu.VMEM_SHARED((ACC,), jnp.float32),
        pltpu.VMEM((CR, 128), jnp.int32),
        pltpu.VMEM((CR, 128), jnp.float32),
        pltpu.VMEM((CR, 128), jnp.int32),
        pltpu.VMEM((CR, 128), jnp.float32),
    ],
    compiler_params=pltpu.CompilerParams(needs_layout_passes=False, use_tc_tiling_on_sc=False),
)()


def _norm_body(src_hbm, dst_hbm, ew_hbm, cdis_hbm, cn_hbm, ci_hbm,
               cdis_v, src_v, dst_v, ew_v, cn_v, ci_v):
    c = lax.axis_index("c")
    s = lax.axis_index("s")
    w = s * 2 + c
    pltpu.sync_copy(cdis_hbm, cdis_v)

    def chunk(ch, _):
        base = w * RPW + ch * CR
        pltpu.sync_copy(src_hbm.at[pl.ds(base, CR)], src_v)
        pltpu.sync_copy(dst_hbm.at[pl.ds(base, CR)], dst_v)
        pltpu.sync_copy(ew_hbm.at[pl.ds(base, CR)], ew_v)

        def grp(t, _):
            j = t // 8
            k = (t % 8) * 16
            ew = ew_v[j, pl.ds(k, 16)]
            sv = src_v[j, pl.ds(k, 16)]
            dv = dst_v[j, pl.ds(k, 16)]
            off = jnp.where(ew < 0.0, jnp.int32(NPAD), jnp.int32(0))
            gs = plsc.load_gather(cdis_v, [sv + off])
            gd = plsc.load_gather(cdis_v, [dv + off])
            cn_v[j, pl.ds(k, 16)] = gs * gd * jnp.abs(ew)
            ci_v[j, pl.ds(k, 16)] = dv + off
            return 0

        lax.fori_loop(0, CR * 8, grp, 0)
        pltpu.sync_copy(cn_v, cn_hbm.at[pl.ds(base, CR)])
        pltpu.sync_copy(ci_v, ci_hbm.at[pl.ds(base, CR)])
        return 0

    lax.fori_loop(0, RPW // CR, chunk, 0)


_norm_call = functools.partial(
    pl.kernel,
    _norm_body,
    out_type=(jax.ShapeDtypeStruct((ROWS, 128), jnp.float32),
              jax.ShapeDtypeStruct((ROWS, 128), jnp.int32)),
    mesh=_MESH,
    scratch_types=[
        pltpu.VMEM((ACC,), jnp.float32),
        pltpu.VMEM((CR, 128), jnp.int32),
        pltpu.VMEM((CR, 128), jnp.int32),
        pltpu.VMEM((CR, 128), jnp.float32),
        pltpu.VMEM((CR, 128), jnp.float32),
        pltpu.VMEM((CR, 128), jnp.int32),
    ],
    compiler_params=pltpu.CompilerParams(needs_layout_passes=False, use_tc_tiling_on_sc=False),
)()


def _agg_body(src_hbm, ci_hbm, cn_hbm, tab_hbm, z_hbm, out_hbm,
              acc, src_v, gi_v, ci_v, cn_v, buf0, buf1, msg0, msg1,
              sem0, sem1, ssem0, ssem1):
    c = lax.axis_index("c")
    s = lax.axis_index("s")
    pltpu.sync_copy(z_hbm, acc.at[pl.ds(s * STRIPE, STRIPE)])
    plsc.subcore_barrier()
    toff = c * jnp.int32(N)

    def scale(buf, msg, j):
        def egrp(g, _):
            cnvec = cn_v[j, pl.ds(g * 16, 16)]
            for l in range(16):
                e = g * 16 + l
                cn = cnvec[l]
                for k in range(4):
                    msg[e, pl.ds(k * 16, 16)] = (
                        buf[e, pl.ds(k * 16, 16)] * cn)
            return 0

        lax.fori_loop(0, 8, egrp, 0)

    def chunk(ch, _):
        base = s * RPS + ch * CR
        pltpu.sync_copy(src_hbm.at[pl.ds(base, CR)], src_v)
        pltpu.sync_copy(ci_hbm.at[pl.ds(base, CR)], ci_v)
        pltpu.sync_copy(cn_hbm.at[pl.ds(base, CR)], cn_v)

        def grp(t, _):
            j = t // 8
            k = (t % 8) * 16
            gi_v[j, pl.ds(k, 16)] = src_v[j, pl.ds(k, 16)] + toff
            return 0

        lax.fori_loop(0, CR * 8, grp, 0)

        pltpu.async_copy(tab_hbm.at[gi_v.at[0]], buf0, sem0)

        def pair(it, _):
            jj = 2 * it
            pltpu.async_copy(tab_hbm.at[gi_v.at[jj + 1]], buf1, sem1)
            pltpu.make_async_copy(tab_hbm.at[gi_v.at[0]], buf0, sem0).wait()

            @pl.when(it > 0)
            def _():
                pltpu.make_async_copy(msg0, acc.at[ci_v.at[0]], ssem0).wait()

            scale(buf0, msg0, jj)
            pltpu.async_copy(msg0, acc.at[ci_v.at[jj]], ssem0, add=True)

            @pl.when(it + 1 < CR // 2)
            def _():
                pltpu.async_copy(tab_hbm.at[gi_v.at[jj + 2]], buf0, sem0)

            pltpu.make_async_copy(tab_hbm.at[gi_v.at[0]], buf1, sem1).wait()

            @pl.when(it > 0)
            def _():
                pltpu.make_async_copy(msg1, acc.at[ci_v.at[0]], ssem1).wait()

            scale(buf1, msg1, jj + 1)
            pltpu.async_copy(msg1, acc.at[ci_v.at[jj + 1]], ssem1, add=True)
            return 0

        lax.fori_loop(0, CR // 2, pair, 0)
        pltpu.make_async_copy(msg0, acc.at[ci_v.at[0]], ssem0).wait()
        pltpu.make_async_copy(msg1, acc.at[ci_v.at[0]], ssem1).wait()
        return 0

    lax.fori_loop(0, RPS // CR, chunk, 0)
    plsc.subcore_barrier()
    pltpu.sync_copy(acc.at[pl.ds(s * STRIPE, STRIPE)],
                    out_hbm.at[pl.ds(c * ACC + s * STRIPE, STRIPE)])


_agg_call = functools.partial(
    pl.kernel,
    _agg_body,
    out_type=jax.ShapeDtypeStruct((2 * ACC, 64), jnp.float32),
    mesh=_MESH,
    scratch_types=[
        pltpu.VMEM_SHARED((ACC, 64), jnp.float32),
        pltpu.VMEM((CR, 128), jnp.int32),
        pltpu.VMEM((CR, 128), jnp.int32),
        pltpu.VMEM((CR, 128), jnp.int32),
        pltpu.VMEM((CR, 128), jnp.float32),
        pltpu.VMEM((128, 64), jnp.float32),
        pltpu.VMEM((128, 64), jnp.float32),
        pltpu.VMEM((128, 64), jnp.float32),
        pltpu.VMEM((128, 64), jnp.float32),
        pltpu.SemaphoreType.DMA,
        pltpu.SemaphoreType.DMA,
        pltpu.SemaphoreType.DMA,
        pltpu.SemaphoreType.DMA,
    ],
    compiler_params=pltpu.CompilerParams(needs_layout_passes=False, use_tc_tiling_on_sc=False),
)()


def _prep_kernel(deg_ref, o_ref):
    d = deg_ref[0] + deg_ref[1]
    o_ref[...] = jnp.where(d > 0.0, lax.rsqrt(jnp.maximum(d, 1e-30)), 0.0)


def _transform_kernel(pa, pb, na, nb, wpa, wpb, wna, wnb, bp, bn, oa, ob):
    dn = (((1,), (0,)), ((), ()))
    px = (lax.dot_general(pa[...], wpa[...], dn, precision=lax.Precision.HIGHEST)
          + lax.dot_general(pb[...], wpb[...], dn, precision=lax.Precision.HIGHEST)
          + bp[...])
    nx = (lax.dot_general(na[...], wna[...], dn, precision=lax.Precision.HIGHEST)
          + lax.dot_general(nb[...], wnb[...], dn, precision=lax.Precision.HIGHEST)
          + bn[...])
    h = jnp.maximum(px, 0.0) - jnp.maximum(nx, 0.0)
    oa[...] = h[:, :64]
    ob[...] = h[:, 64:]


def _final_kernel(pa, pb, na, nb, wpa, wpb, wna, wnb, bp, bn, bat, gam, bet,
                  o_ref, acc, cnt):
    i = pl.program_id(0)

    @pl.when(i == 0)
    def _():
        acc[...] = jnp.zeros_like(acc)
        cnt[...] = jnp.zeros_like(cnt)

    dn = (((1,), (0,)), ((), ()))
    dnp = (((0,), (0,)), ((), ()))
    px = (lax.dot_general(pa[...], wpa[...], dn, precision=lax.Precision.HIGHEST)
          + lax.dot_general(pb[...], wpb[...], dn, precision=lax.Precision.HIGHEST)
          + bp[...])
    nx = (lax.dot_general(na[...], wna[...], dn, precision=lax.Precision.HIGHEST)
          + lax.dot_general(nb[...], wnb[...], dn, precision=lax.Precision.HIGHEST)
          + bn[...])
    h = jnp.maximum(px, 0.0) - jnp.maximum(nx, 0.0)
    oh = (bat[...] == lax.broadcasted_iota(jnp.int32, (400, G), 1
                                           ).astype(jnp.float32)
          ).astype(jnp.float32)
    acc[...] += lax.dot_general(oh, h, dnp, precision=lax.Precision.HIGHEST)
    cnt[...] += lax.dot_general(oh, jnp.ones_like(h), dnp,
                                precision=lax.Precision.HIGHEST)

    @pl.when(i == pl.num_programs(0) - 1)
    def _():
        pooled = acc[...] / jnp.maximum(cnt[...], 1.0)
        mu = jnp.mean(pooled, axis=-1, keepdims=True)
        dev = pooled - mu
        var = jnp.mean(dev * dev, axis=-1, keepdims=True)
        o_ref[...] = dev * lax.rsqrt(var + EPS) * gam[...] + bet[...]


def _transform(aggf, Wp, bp, Wn, bn):
    pa = aggf[0:N]
    na = aggf[NPAD:NPAD + N]
    pb = aggf[ACC:ACC + N]
    nb = aggf[ACC + NPAD:ACC + NPAD + N]
    bn_ = 400
    grid = N // bn_
    rspec = pl.BlockSpec((bn_, 64), lambda i: (i, 0))
    wspec = pl.BlockSpec((64, 128), lambda i: (0, 0))
    bspec = pl.BlockSpec((1, 128), lambda i: (0, 0))
    return pl.pallas_call(
        _transform_kernel,
        grid=(grid,),
        in_specs=[rspec, rspec, rspec, rspec,
                  wspec, wspec, wspec, wspec, bspec, bspec],
        out_specs=[pl.BlockSpec((bn_, 64), lambda i: (i, 0))] * 2,
        out_shape=[jax.ShapeDtypeStruct((N, 64), jnp.float32)] * 2,
        compiler_params=pltpu.CompilerParams(
            dimension_semantics=("arbitrary",)),
    )(pa, pb, na, nb, Wp[:64], Wp[64:], Wn[:64], Wn[64:],
      bp.reshape(1, 128), bn.reshape(1, 128))


def _final(aggf, Wp, bp, Wn, bn, batf, gamma, beta):
    pa = aggf[0:N]
    na = aggf[NPAD:NPAD + N]
    pb = aggf[ACC:ACC + N]
    nb = aggf[ACC + NPAD:ACC + NPAD + N]
    bn_ = 400
    grid = N // bn_
    rspec = pl.BlockSpec((bn_, 64), lambda i: (i, 0))
    wspec = pl.BlockSpec((64, 128), lambda i: (0, 0))
    bspec = pl.BlockSpec((1, 128), lambda i: (0, 0))
    return pl.pallas_call(
        _final_kernel,
        grid=(grid,),
        in_specs=[rspec, rspec, rspec, rspec,
                  wspec, wspec, wspec, wspec, bspec, bspec,
                  pl.BlockSpec((bn_, 1), lambda i: (i, 0)),
                  bspec, bspec],
        out_specs=pl.BlockSpec((G, 128), lambda i: (0, 0)),
        out_shape=jax.ShapeDtypeStruct((G, 128), jnp.float32),
        scratch_shapes=[pltpu.VMEM((G, 128), jnp.float32),
                        pltpu.VMEM((G, 128), jnp.float32)],
        compiler_params=pltpu.CompilerParams(
            dimension_semantics=("arbitrary",)),
    )(pa, pb, na, nb, Wp[:64], Wp[64:], Wn[:64], Wn[64:],
      bp.reshape(1, 128), bn.reshape(1, 128), batf,
      gamma.reshape(1, 128), beta.reshape(1, 128))


def kernel(x, edge_index, edge_weight, batch,
           Wp0, bp0, Wp1, bp1, Wn0, bn0, Wn1, bn1, gamma, beta):
    ar = jnp.arange(N, dtype=jnp.int32)
    padi = jnp.arange(PADCNT, dtype=jnp.int32) % N
    src = jnp.concatenate([edge_index[0], ar, ar, padi]).reshape(ROWS, 128)
    dst = jnp.concatenate([edge_index[1], ar, ar, padi]).reshape(ROWS, 128)
    ew = jnp.concatenate([
        edge_weight, jnp.ones((N,), jnp.float32),
        -jnp.ones((N,), jnp.float32), jnp.zeros((PADCNT,), jnp.float32),
    ]).reshape(ROWS, 128)
    zdeg = jnp.zeros((STRIPE,), jnp.float32)
    zagg = jnp.zeros((STRIPE, 64), jnp.float32)

    degf = _deg_call(dst, ew, zdeg)
    cdis = pl.pallas_call(
        _prep_kernel,
        out_shape=jax.ShapeDtypeStruct((ACC // 128, 128), jnp.float32),
    )(degf.reshape(2, ACC // 128, 128)).reshape(ACC)
    cn2d, ci2d = _norm_call(src, dst, ew, cdis)

    tab1 = jnp.concatenate([x[:, :64], x[:, 64:]], axis=0)
    agg1 = _agg_call(src, ci2d, cn2d, tab1, zagg)
    h1a, h1b = _transform(agg1, Wp0, bp0, Wn0, bn0)

    tab2 = jnp.concatenate([h1a, h1b], axis=0)
    agg2 = _agg_call(src, ci2d, cn2d, tab2, zagg)

    batf = batch.astype(jnp.float32).reshape(N, 1)
    return _final(agg2, Wp1, bp1, Wn1, bn1, batf, gamma, beta)
